# Initial kernel scaffold; baseline (speedup 1.0000x reference)
import jax
import jax.numpy as jnp
from jax.experimental import pallas as pl


def kernel(entity_emb, edge_index, edge_type, edge_emb, q_w, k_w, mess_dropout):
    raise NotImplementedError("write your pallas kernel here")


# speedup vs baseline: 3.4821x; 3.4821x over previous
import functools

import jax
import jax.numpy as jnp
from jax import lax
from jax.experimental import pallas as pl
from jax.experimental.pallas import tpu as pltpu
from jax.experimental.pallas import tpu_sc as plsc

N = 10000
E = 320000
D = 128
R = 16
EPS = 1e-12

NP = 10240
NC = 2
NS = 16
NW = NC * NS
EW = E // NW
CH = 80
NCHUNK = EW // CH
GP = CH // 16
RT = NP // NS
NEG = -3.4e38

_SC_PARAMS = pltpu.CompilerParams(needs_layout_passes=False)


def _mesh():
    return plsc.VectorSubcoreMesh(
        core_axis_name="c", subcore_axis_name="s", num_cores=NC, num_subcores=NS
    )


def _wid(c, s):
    return c * NS + s


def _seg_runs(hbuf, vbuf, h, vals, op):
    hs, vs = plsc.sort_key_val(h, vals)
    hbuf[pl.ds(16, 16)] = hs
    for sh in (1, 2, 4, 8):
        vbuf[pl.ds(16, 16)] = vs
        shh = hbuf[pl.ds(16 + sh, 16)]
        sha = vbuf[pl.ds(16 + sh, 16)]
        same = shh == hs
        if op == "max":
            vs = jnp.maximum(vs, jnp.where(same, sha, NEG))
        else:
            vs = vs + jnp.where(same, sha, 0.0)
    prev = hbuf[pl.ds(15, 16)]
    first = prev != hs
    return hs, vs, first


def _runs_from_sorted(hbuf, vbuf, hs, vs, op):
    for sh in (1, 2, 4, 8):
        vbuf[pl.ds(16, 16)] = vs
        shh = hbuf[pl.ds(16 + sh, 16)]
        sha = vbuf[pl.ds(16 + sh, 16)]
        same = shh == hs
        if op == "max":
            vs = jnp.maximum(vs, jnp.where(same, sha, NEG))
        else:
            vs = vs + jnp.where(same, sha, 0.0)
    return vs


def _init_hbuf(hbuf):
    neg1 = jnp.full((16,), -1, jnp.int32)
    hbuf[pl.ds(0, 16)] = neg1
    hbuf[pl.ds(32, 16)] = neg1


def _fill(ref, value, n16):
    v = jnp.full((16,), value, ref.dtype)

    def body(i, _):
        ref[pl.ds(i * 16, 16)] = v
        return 0

    lax.fori_loop(0, n16, body, 0)


def _combine_16(shared, priv, redbuf, outbuf, out_hbm, c, s, op):
    pltpu.sync_copy(priv, shared.at[s])
    plsc.subcore_barrier()
    for k in range(NS):
        pltpu.sync_copy(shared.at[k, pl.ds(s * RT, RT)], redbuf.at[k])

    def body(j, _):
        acc = redbuf[0, pl.ds(j * 16, 16)]
        for k in range(1, NS):
            v = redbuf[k, pl.ds(j * 16, 16)]
            acc = jnp.maximum(acc, v) if op == "max" else acc + v
        outbuf[pl.ds(j * 16, 16)] = acc
        return 0

    lax.fori_loop(0, RT // 16, body, 0)
    pltpu.sync_copy(outbuf, out_hbm.at[c, pl.ds(s * RT, RT)])


def _stage_a(agg, edge_emb, q_w, k_w):
    nb = 256
    gi = NP // nb

    def body(agg_ref, emb_ref, q_ref, k_ref, p_ref, t2_ref):
        r = pl.program_id(1)
        a = agg_ref[...]

        @pl.when(r == 0)
        def _():
            p_ref[...] = jnp.dot(
                a, q_ref[...], preferred_element_type=jnp.float32
            )

        m = a * emb_ref[pl.ds(r, 1), :]
        t2_ref[...] = jnp.tanh(
            jnp.dot(m, k_ref[...], preferred_element_type=jnp.float32)
        )

    return pl.pallas_call(
        body,
        grid=(gi, R),
        in_specs=[
            pl.BlockSpec((nb, D), lambda i, r: (i, 0)),
            pl.BlockSpec((R, D), lambda i, r: (0, 0)),
            pl.BlockSpec((D, D), lambda i, r: (0, 0)),
            pl.BlockSpec((D, D), lambda i, r: (0, 0)),
        ],
        out_specs=[
            pl.BlockSpec((nb, D), lambda i, r: (i, 0)),
            pl.BlockSpec((nb, D), lambda i, r: (r * gi + i, 0)),
        ],
        out_shape=[
            jax.ShapeDtypeStruct((NP, D), jnp.float32),
            jax.ShapeDtypeStruct((R * NP, D), jnp.float32),
        ],
    )(agg, edge_emb, q_w, k_w)


def _make_stage_b():
    @functools.partial(
        pl.kernel,
        mesh=_mesh(),
        compiler_params=_SC_PARAMS,
        out_type=[
            jax.ShapeDtypeStruct((E,), jnp.float32),
            jax.ShapeDtypeStruct((NC, NP), jnp.float32),
        ],
        scratch_types=[
            pltpu.VMEM((CH,), jnp.int32),
            pltpu.VMEM((CH,), jnp.int32),
            pltpu.VMEM((CH,), jnp.int32),
            pltpu.VMEM((CH,), jnp.int32),
            pltpu.VMEM((CH, D), jnp.float32),
            pltpu.VMEM((CH, D), jnp.float32),
            pltpu.VMEM((CH,), jnp.float32),
            pltpu.VMEM((NP,), jnp.float32),
            pltpu.VMEM((48,), jnp.int32),
            pltpu.VMEM((48,), jnp.float32),
            pltpu.VMEM_SHARED((NS, NP), jnp.float32),
            pltpu.VMEM((NS, RT), jnp.float32),
            pltpu.VMEM((RT,), jnp.float32),
            pltpu.SemaphoreType.DMA,
            pltpu.SemaphoreType.DMA,
        ],
    )
    def stage_b(p_hbm, t2_hbm, head_hbm, tail_hbm, type_hbm, att_hbm, max_hbm,
                headv, tailv, typev, idxv, pbuf, tbuf, attv, maxpriv,
                hbuf, vbuf, shared, redbuf, outbuf, sem1, sem2):
        c = lax.axis_index("c")
        s = lax.axis_index("s")
        base0 = _wid(c, s) * EW
        _init_hbuf(hbuf)
        _fill(maxpriv, NEG, NP // 16)
        lanes = lax.iota(jnp.int32, 16)

        def chunk(j, _):
            base = base0 + j * CH
            pltpu.sync_copy(head_hbm.at[pl.ds(base, CH)], headv)
            pltpu.sync_copy(tail_hbm.at[pl.ds(base, CH)], tailv)
            pltpu.sync_copy(type_hbm.at[pl.ds(base, CH)], typev)
            for g in range(GP):
                idxv[pl.ds(g * 16, 16)] = (
                    typev[pl.ds(g * 16, 16)] * NP + tailv[pl.ds(g * 16, 16)]
                )
            cp1 = pltpu.async_copy(p_hbm.at[headv], pbuf, sem1)
            cp2 = pltpu.async_copy(t2_hbm.at[idxv], tbuf, sem2)
            cp1.wait()
            cp2.wait()
            for g in range(GP):
                def edot(jj, att_vec):
                    e = g * 16 + jj
                    acc = pbuf[e, pl.ds(0, 16)] * tbuf[e, pl.ds(0, 16)]
                    for k in range(1, 8):
                        acc = acc + (
                            pbuf[e, pl.ds(k * 16, 16)]
                            * tbuf[e, pl.ds(k * 16, 16)]
                        )
                    sdot = jnp.sum(acc)
                    return jnp.where(lanes == jj, sdot, att_vec)

                att_vec = lax.fori_loop(0, 16, edot, jnp.zeros((16,), jnp.float32))
                attv[pl.ds(g * 16, 16)] = att_vec
                h = headv[pl.ds(g * 16, 16)]
                hs, vs, first = _seg_runs(hbuf, vbuf, h, att_vec, "max")
                old = plsc.load_gather(maxpriv, [hs])
                plsc.store_scatter(maxpriv, [hs], jnp.maximum(old, vs), mask=first)
            pltpu.sync_copy(attv, att_hbm.at[pl.ds(base, CH)])
            return 0

        lax.fori_loop(0, NCHUNK, chunk, 0)
        _combine_16(shared, maxpriv, redbuf, outbuf, max_hbm, c, s, "max")

    return stage_b


def _make_stage_c():
    @functools.partial(
        pl.kernel,
        mesh=_mesh(),
        compiler_params=_SC_PARAMS,
        out_type=[
            jax.ShapeDtypeStruct((E,), jnp.float32),
            jax.ShapeDtypeStruct((NC, NP), jnp.float32),
        ],
        scratch_types=[
            pltpu.VMEM((CH,), jnp.int32),
            pltpu.VMEM((CH,), jnp.float32),
            pltpu.VMEM((CH,), jnp.float32),
            pltpu.VMEM((NP,), jnp.float32),
            pltpu.VMEM((NP,), jnp.float32),
            pltpu.VMEM((NP,), jnp.float32),
            pltpu.VMEM((48,), jnp.int32),
            pltpu.VMEM((48,), jnp.float32),
            pltpu.VMEM_SHARED((NS, NP), jnp.float32),
            pltpu.VMEM((NS, RT), jnp.float32),
            pltpu.VMEM((RT,), jnp.float32),
        ],
    )
    def stage_c(att_hbm, head_hbm, max_hbm, ex_hbm, den_hbm,
                headv, attv, exv, m0buf, m1buf, denompriv,
                hbuf, vbuf, shared, redbuf, outbuf):
        c = lax.axis_index("c")
        s = lax.axis_index("s")
        base0 = _wid(c, s) * EW
        _init_hbuf(hbuf)
        pltpu.sync_copy(max_hbm.at[0], m0buf)
        pltpu.sync_copy(max_hbm.at[1], m1buf)

        def mx(i, _):
            m0buf[pl.ds(i * 16, 16)] = jnp.maximum(
                m0buf[pl.ds(i * 16, 16)], m1buf[pl.ds(i * 16, 16)]
            )
            return 0

        lax.fori_loop(0, NP // 16, mx, 0)
        _fill(denompriv, 0.0, NP // 16)

        def chunk(j, _):
            base = base0 + j * CH
            pltpu.sync_copy(head_hbm.at[pl.ds(base, CH)], headv)
            pltpu.sync_copy(att_hbm.at[pl.ds(base, CH)], attv)
            for g in range(GP):
                h = headv[pl.ds(g * 16, 16)]
                a = attv[pl.ds(g * 16, 16)]
                gm = plsc.load_gather(m0buf, [h])
                ex = jnp.exp(a - gm)
                exv[pl.ds(g * 16, 16)] = ex
                hs, es = plsc.sort_key_val(h, ex)
                hbuf[pl.ds(16, 16)] = hs
                esr = _runs_from_sorted(hbuf, vbuf, hs, es, "add")
                prev = hbuf[pl.ds(15, 16)]
                first = prev != hs
                oldd = plsc.load_gather(denompriv, [hs])
                plsc.store_scatter(denompriv, [hs], oldd + esr, mask=first)
            pltpu.sync_copy(exv, ex_hbm.at[pl.ds(base, CH)])
            return 0

        lax.fori_loop(0, NCHUNK, chunk, 0)
        _combine_16(shared, denompriv, redbuf, outbuf, den_hbm, c, s, "add")

    return stage_c


def _rsqrt_newton(x):
    i = lax.bitcast_convert_type(x, jnp.int32)
    i = 0x5F3759DF - lax.shift_right_logical(i, 1)
    y = lax.bitcast_convert_type(i, jnp.float32)
    for _ in range(3):
        y = y * (1.5 - 0.5 * x * y * y)
    return y


def _make_stage_d():
    @functools.partial(
        pl.kernel,
        mesh=_mesh(),
        compiler_params=_SC_PARAMS,
        out_type=jax.ShapeDtypeStruct((NC, NP, D), jnp.float32),
        scratch_types=[
            pltpu.VMEM((CH,), jnp.int32),
            pltpu.VMEM((CH,), jnp.int32),
            pltpu.VMEM((96,), jnp.int32),
            pltpu.VMEM((CH,), jnp.float32),
            pltpu.VMEM((96,), jnp.float32),
            pltpu.VMEM((NP,), jnp.float32),
            pltpu.VMEM((NP,), jnp.float32),
            pltpu.VMEM((R, D), jnp.float32),
            pltpu.VMEM((CH, D), jnp.float32),
            pltpu.VMEM((CH, D), jnp.float32),
            pltpu.VMEM((16, D), jnp.float32),
            pltpu.VMEM_SHARED((NP, D), jnp.float32),
            pltpu.SemaphoreType.DMA,
        ],
    )
    def stage_d(ex_hbm, head_hbm, tail_hbm, type_hbm, den_hbm, emb_hbm,
                agg_hbm, sum_hbm,
                headv, tailv, typev, exvc, attnv, denl, tmpl, embv,
                abuf, msgbuf, zbuf, acc, sem):
        c = lax.axis_index("c")
        s = lax.axis_index("s")
        base0 = _wid(c, s) * EW
        pltpu.sync_copy(den_hbm.at[0], denl)
        pltpu.sync_copy(den_hbm.at[1], tmpl)

        def ad(i, _):
            denl[pl.ds(i * 16, 16)] = (
                denl[pl.ds(i * 16, 16)] + tmpl[pl.ds(i * 16, 16)]
            )
            return 0

        lax.fori_loop(0, NP // 16, ad, 0)
        pltpu.sync_copy(emb_hbm, embv)
        z16 = jnp.zeros((16,), jnp.float32)
        for zr in range(16):
            for zk in range(8):
                zbuf[zr, pl.ds(zk * 16, 16)] = z16

        def zinit(i, _):
            pltpu.sync_copy(zbuf, acc.at[pl.ds(s * RT + i * 16, 16)])
            return 0

        lax.fori_loop(0, RT // 16, zinit, 0)
        plsc.subcore_barrier()

        def chunk(j, _):
            base = base0 + j * CH
            pltpu.sync_copy(head_hbm.at[pl.ds(base, CH)], headv)
            pltpu.sync_copy(tail_hbm.at[pl.ds(base, CH)], tailv)
            pltpu.sync_copy(type_hbm.at[pl.ds(base, CH)], typev.at[pl.ds(0, CH)])
            pltpu.sync_copy(ex_hbm.at[pl.ds(base, CH)], exvc)
            pltpu.async_copy(agg_hbm.at[tailv], abuf, sem).wait()
            for g in range(GP):
                h = headv[pl.ds(g * 16, 16)]
                ex = exvc[pl.ds(g * 16, 16)]
                dg = plsc.load_gather(denl, [h])
                attnv[pl.ds(g * 16, 16)] = ex / jnp.maximum(dg, EPS)

            def edge(e, _):
                ty = typev[pl.ds(e, 16)][0]
                at = attnv[pl.ds(e, 16)][0]
                for k in range(8):
                    msgbuf[e, pl.ds(k * 16, 16)] = (
                        abuf[e, pl.ds(k * 16, 16)]
                        * embv[ty, pl.ds(k * 16, 16)]
                        * at
                    )
                return 0

            lax.fori_loop(0, CH, edge, 0)
            pltpu.sync_copy(msgbuf, acc.at[headv], add=True)
            return 0

        lax.fori_loop(0, NCHUNK, chunk, 0)
        plsc.subcore_barrier()
        pltpu.sync_copy(acc.at[pl.ds(s * RT, RT)], sum_hbm.at[c, pl.ds(s * RT, RT)])

    return stage_d


def _stage_mean_norm(sumpart):
    def body(s0_ref, s1_ref, agg_ref):
        summed = s0_ref[0] + s1_ref[0]
        nrm = jnp.sqrt(jnp.sum(summed * summed, axis=1, keepdims=True))
        agg_ref[...] = summed / jnp.maximum(nrm, EPS)

    nb = 1024
    return pl.pallas_call(
        body,
        grid=(NP // nb,),
        in_specs=[
            pl.BlockSpec((1, nb, D), lambda i: (0, i, 0)),
            pl.BlockSpec((1, nb, D), lambda i: (1, i, 0)),
        ],
        out_specs=pl.BlockSpec((nb, D), lambda i: (i, 0)),
        out_shape=jax.ShapeDtypeStruct((NP, D), jnp.float32),
    )(sumpart, sumpart)


def _stage_final(agg1, agg2, ent):
    def body(a1, a2, e, o):
        o[...] = a1[...] + a2[...] + 2.0 * e[...]

    nb = 256
    return pl.pallas_call(
        body,
        grid=(NP // nb,),
        in_specs=[pl.BlockSpec((nb, D), lambda i: (i, 0))] * 3,
        out_specs=pl.BlockSpec((nb, D), lambda i: (i, 0)),
        out_shape=jax.ShapeDtypeStruct((NP, D), jnp.float32),
    )(agg1, agg2, ent)


@functools.cache
def _build():
    return _make_stage_b(), _make_stage_c(), _make_stage_d()


def kernel(entity_emb, edge_index, edge_type, edge_emb, q_w, k_w, mess_dropout):
    stage_b, stage_c, stage_d = _build()
    head = edge_index[0]
    tail = edge_index[1]
    etype = edge_type.astype(jnp.int32)
    ent_p = jnp.pad(entity_emb, ((0, NP - N), (0, 0)))
    agg = ent_p
    aggs = []
    for _hop in range(2):
        p_arr, t2f = _stage_a(agg, edge_emb, q_w, k_w)
        att, maxpart = stage_b(p_arr, t2f, head, tail, etype)
        ex, denpart = stage_c(att, head, maxpart)
        sumpart = stage_d(ex, head, tail, etype, denpart, edge_emb, agg)
        aggn = _stage_mean_norm(sumpart)
        aggs.append(aggn)
        agg = aggn
    kg = _stage_final(aggs[0], aggs[1], ent_p)
    return kg[:N]


# speedup vs baseline: 6.7363x; 1.9345x over previous
import functools

import jax
import jax.numpy as jnp
from jax import lax
from jax.experimental import pallas as pl
from jax.experimental.pallas import tpu as pltpu
from jax.experimental.pallas import tpu_sc as plsc

N = 10000
E = 320000
D = 128
R = 16
EPS = 1e-12

NP = 10240
NC = 2
NS = 16
NW = NC * NS
EW = E // NW
CH = 80
NCHUNK = EW // CH
GP = CH // 16
RT = NP // NS
NEG = -3.4e38

_SC_PARAMS = pltpu.CompilerParams(needs_layout_passes=False)


def _mesh():
    return plsc.VectorSubcoreMesh(
        core_axis_name="c", subcore_axis_name="s", num_cores=NC, num_subcores=NS
    )


def _wid(c, s):
    return c * NS + s


def _seg_runs(hbuf, vbuf, h, vals, op):
    hs, vs = plsc.sort_key_val(h, vals)
    hbuf[pl.ds(16, 16)] = hs
    for sh in (1, 2, 4, 8):
        vbuf[pl.ds(16, 16)] = vs
        shh = hbuf[pl.ds(16 + sh, 16)]
        sha = vbuf[pl.ds(16 + sh, 16)]
        same = shh == hs
        if op == "max":
            vs = jnp.maximum(vs, jnp.where(same, sha, NEG))
        else:
            vs = vs + jnp.where(same, sha, 0.0)
    prev = hbuf[pl.ds(15, 16)]
    first = prev != hs
    return hs, vs, first


def _runs_from_sorted(hbuf, vbuf, hs, vs, op):
    for sh in (1, 2, 4, 8):
        vbuf[pl.ds(16, 16)] = vs
        shh = hbuf[pl.ds(16 + sh, 16)]
        sha = vbuf[pl.ds(16 + sh, 16)]
        same = shh == hs
        if op == "max":
            vs = jnp.maximum(vs, jnp.where(same, sha, NEG))
        else:
            vs = vs + jnp.where(same, sha, 0.0)
    return vs


def _init_hbuf(hbuf):
    neg1 = jnp.full((16,), -1, jnp.int32)
    hbuf[pl.ds(0, 16)] = neg1
    hbuf[pl.ds(32, 16)] = neg1


def _fill(ref, value, n16):
    v = jnp.full((16,), value, ref.dtype)

    def body(i, _):
        ref[pl.ds(i * 16, 16)] = v
        return 0

    lax.fori_loop(0, n16, body, 0)


def _combine_16(shared, priv, redbuf, outbuf, out_hbm, c, s, op):
    pltpu.sync_copy(priv, shared.at[s])
    plsc.subcore_barrier()
    for k in range(NS):
        pltpu.sync_copy(shared.at[k, pl.ds(s * RT, RT)], redbuf.at[k])

    def body(j, _):
        acc = redbuf[0, pl.ds(j * 16, 16)]
        for k in range(1, NS):
            v = redbuf[k, pl.ds(j * 16, 16)]
            acc = jnp.maximum(acc, v) if op == "max" else acc + v
        outbuf[pl.ds(j * 16, 16)] = acc
        return 0

    lax.fori_loop(0, RT // 16, body, 0)
    pltpu.sync_copy(outbuf, out_hbm.at[c, pl.ds(s * RT, RT)])


def _stage_a(agg, edge_emb, q_w, k_w):
    nb = 1024
    gi = NP // nb

    def body(agg_ref, emb_ref, q_ref, k_ref, p_ref, t2_ref):
        r = pl.program_id(1)
        a = agg_ref[...]

        @pl.when(r == 0)
        def _():
            p_ref[...] = jnp.dot(
                a, q_ref[...], preferred_element_type=jnp.float32
            )

        m = a * emb_ref[pl.ds(r, 1), :]
        t2_ref[...] = jnp.tanh(
            jnp.dot(m, k_ref[...], preferred_element_type=jnp.float32)
        )

    return pl.pallas_call(
        body,
        grid=(gi, R),
        in_specs=[
            pl.BlockSpec((nb, D), lambda i, r: (i, 0)),
            pl.BlockSpec((R, D), lambda i, r: (0, 0)),
            pl.BlockSpec((D, D), lambda i, r: (0, 0)),
            pl.BlockSpec((D, D), lambda i, r: (0, 0)),
        ],
        out_specs=[
            pl.BlockSpec((nb, D), lambda i, r: (i, 0)),
            pl.BlockSpec((nb, D), lambda i, r: (r * gi + i, 0)),
        ],
        out_shape=[
            jax.ShapeDtypeStruct((NP, D), jnp.float32),
            jax.ShapeDtypeStruct((R * NP, D), jnp.float32),
        ],
    )(agg, edge_emb, q_w, k_w)


def _make_stage_b():
    @functools.partial(
        pl.kernel,
        mesh=_mesh(),
        compiler_params=_SC_PARAMS,
        out_type=[
            jax.ShapeDtypeStruct((E,), jnp.float32),
            jax.ShapeDtypeStruct((NC, NP), jnp.float32),
        ],
        scratch_types=[
            pltpu.VMEM((EW,), jnp.int32),
            pltpu.VMEM((EW,), jnp.int32),
            pltpu.VMEM((EW,), jnp.int32),
            pltpu.VMEM((EW,), jnp.float32),
            pltpu.VMEM((2, CH, D), jnp.float32),
            pltpu.VMEM((2, CH, D), jnp.float32),
            pltpu.VMEM((NP,), jnp.float32),
            pltpu.VMEM((48,), jnp.int32),
            pltpu.VMEM((48,), jnp.float32),
            pltpu.VMEM_SHARED((NS, NP), jnp.float32),
            pltpu.VMEM((NS, RT), jnp.float32),
            pltpu.VMEM((RT,), jnp.float32),
            pltpu.SemaphoreType.DMA,
            pltpu.SemaphoreType.DMA,
            pltpu.SemaphoreType.DMA,
            pltpu.SemaphoreType.DMA,
        ],
    )
    def stage_b(p_hbm, t2_hbm, head_hbm, tail_hbm, type_hbm, att_hbm, max_hbm,
                headall, idxall, typeall, attall, pbuf, tbuf, maxpriv,
                hbuf, vbuf, shared, redbuf, outbuf, semp0, semp1, semt0, semt1):
        c = lax.axis_index("c")
        s = lax.axis_index("s")
        base0 = _wid(c, s) * EW
        _init_hbuf(hbuf)
        _fill(maxpriv, NEG, NP // 16)
        lanes = lax.iota(jnp.int32, 16)
        pltpu.sync_copy(head_hbm.at[pl.ds(base0, EW)], headall)
        pltpu.sync_copy(tail_hbm.at[pl.ds(base0, EW)], idxall)
        pltpu.sync_copy(type_hbm.at[pl.ds(base0, EW)], typeall)

        def mkidx(i, _):
            idxall[pl.ds(i * 16, 16)] = (
                typeall[pl.ds(i * 16, 16)] * NP + idxall[pl.ds(i * 16, 16)]
            )
            return 0

        lax.fori_loop(0, EW // 16, mkidx, 0)
        semp = (semp0, semp1)
        semt = (semt0, semt1)

        def fire(j, slot):
            pltpu.async_copy(
                p_hbm.at[headall.at[pl.ds(j * CH, CH)]], pbuf.at[slot], semp[slot]
            )
            pltpu.async_copy(
                t2_hbm.at[idxall.at[pl.ds(j * CH, CH)]], tbuf.at[slot], semt[slot]
            )

        def drain(slot):
            pltpu.make_async_copy(
                p_hbm.at[pl.ds(0, CH)], pbuf.at[slot], semp[slot]
            ).wait()
            pltpu.make_async_copy(
                t2_hbm.at[pl.ds(0, CH)], tbuf.at[slot], semt[slot]
            ).wait()

        def compute(j, slot):
            pb = pbuf.at[slot]
            tb = tbuf.at[slot]
            for g in range(GP):
                def edot(jj, att_vec):
                    e = g * 16 + jj
                    acc = pb[e, pl.ds(0, 16)] * tb[e, pl.ds(0, 16)]
                    for k in range(1, 8):
                        acc = acc + (
                            pb[e, pl.ds(k * 16, 16)] * tb[e, pl.ds(k * 16, 16)]
                        )
                    sdot = jnp.sum(acc)
                    return jnp.where(lanes == jj, sdot, att_vec)

                att_vec = lax.fori_loop(
                    0, 16, edot, jnp.zeros((16,), jnp.float32)
                )
                attall[pl.ds(j * CH + g * 16, 16)] = att_vec
                h = headall[pl.ds(j * CH + g * 16, 16)]
                hs, vs, first = _seg_runs(hbuf, vbuf, h, att_vec, "max")
                old = plsc.load_gather(maxpriv, [hs])
                plsc.store_scatter(
                    maxpriv, [hs], jnp.maximum(old, vs), mask=first
                )

        fire(0, 0)
        fire(1, 1)

        def pair(jj, _):
            for slot in range(2):
                j = jj * 2 + slot
                drain(slot)
                compute(j, slot)

                @pl.when(j + 2 < NCHUNK)
                def _():
                    fire(j + 2, slot)
            return 0

        lax.fori_loop(0, NCHUNK // 2, pair, 0)
        drain(0)
        compute(NCHUNK - 1, 0)
        pltpu.sync_copy(attall, att_hbm.at[pl.ds(base0, EW)])
        _combine_16(shared, maxpriv, redbuf, outbuf, max_hbm, c, s, "max")

    return stage_b


def _make_stage_c():
    @functools.partial(
        pl.kernel,
        mesh=_mesh(),
        compiler_params=_SC_PARAMS,
        out_type=[
            jax.ShapeDtypeStruct((E,), jnp.float32),
            jax.ShapeDtypeStruct((NC, NP), jnp.float32),
        ],
        scratch_types=[
            pltpu.VMEM((EW,), jnp.int32),
            pltpu.VMEM((EW,), jnp.float32),
            pltpu.VMEM((NP,), jnp.float32),
            pltpu.VMEM((NP,), jnp.float32),
            pltpu.VMEM((NP,), jnp.float32),
            pltpu.VMEM((48,), jnp.int32),
            pltpu.VMEM((48,), jnp.float32),
            pltpu.VMEM_SHARED((NS, NP), jnp.float32),
            pltpu.VMEM((NS, RT), jnp.float32),
            pltpu.VMEM((RT,), jnp.float32),
        ],
    )
    def stage_c(att_hbm, head_hbm, max_hbm, ex_hbm, den_hbm,
                headall, attall, m0buf, m1buf, denompriv,
                hbuf, vbuf, shared, redbuf, outbuf):
        c = lax.axis_index("c")
        s = lax.axis_index("s")
        base0 = _wid(c, s) * EW
        _init_hbuf(hbuf)
        pltpu.sync_copy(max_hbm.at[0], m0buf)
        pltpu.sync_copy(max_hbm.at[1], m1buf)
        pltpu.sync_copy(head_hbm.at[pl.ds(base0, EW)], headall)
        pltpu.sync_copy(att_hbm.at[pl.ds(base0, EW)], attall)

        def mx(i, _):
            m0buf[pl.ds(i * 16, 16)] = jnp.maximum(
                m0buf[pl.ds(i * 16, 16)], m1buf[pl.ds(i * 16, 16)]
            )
            return 0

        lax.fori_loop(0, NP // 16, mx, 0)
        _fill(denompriv, 0.0, NP // 16)

        def group(g, _):
            h = headall[pl.ds(g * 16, 16)]
            a = attall[pl.ds(g * 16, 16)]
            gm = plsc.load_gather(m0buf, [h])
            ex = jnp.exp(a - gm)
            attall[pl.ds(g * 16, 16)] = ex
            hs, es = plsc.sort_key_val(h, ex)
            hbuf[pl.ds(16, 16)] = hs
            esr = _runs_from_sorted(hbuf, vbuf, hs, es, "add")
            prev = hbuf[pl.ds(15, 16)]
            first = prev != hs
            oldd = plsc.load_gather(denompriv, [hs])
            plsc.store_scatter(denompriv, [hs], oldd + esr, mask=first)
            return 0

        lax.fori_loop(0, EW // 16, group, 0)
        pltpu.sync_copy(attall, ex_hbm.at[pl.ds(base0, EW)])
        _combine_16(shared, denompriv, redbuf, outbuf, den_hbm, c, s, "add")

    return stage_c


def _rsqrt_newton(x):
    i = lax.bitcast_convert_type(x, jnp.int32)
    i = 0x5F3759DF - lax.shift_right_logical(i, 1)
    y = lax.bitcast_convert_type(i, jnp.float32)
    for _ in range(3):
        y = y * (1.5 - 0.5 * x * y * y)
    return y


def _make_stage_d():
    @functools.partial(
        pl.kernel,
        mesh=_mesh(),
        compiler_params=_SC_PARAMS,
        out_type=jax.ShapeDtypeStruct((NC, NP, D), jnp.float32),
        scratch_types=[
            pltpu.VMEM((EW,), jnp.int32),
            pltpu.VMEM((2, CH), jnp.int32),
            pltpu.VMEM((2, 96), jnp.int32),
            pltpu.VMEM((2, CH), jnp.float32),
            pltpu.VMEM((96,), jnp.float32),
            pltpu.VMEM((NP,), jnp.float32),
            pltpu.VMEM((R, D), jnp.float32),
            pltpu.VMEM((2, CH, D), jnp.float32),
            pltpu.VMEM((16, D), jnp.float32),
            pltpu.VMEM_SHARED((NP, D), jnp.float32),
            pltpu.SemaphoreType.DMA,
            pltpu.SemaphoreType.DMA,
            pltpu.SemaphoreType.DMA,
            pltpu.SemaphoreType.DMA,
        ],
    )
    def stage_d(ex_hbm, head_hbm, tail_hbm, type_hbm, den_hbm, emb_hbm,
                agg_hbm, sum_hbm,
                tailall, headv, typev, exv, attnv, denl, embv,
                abuf, zbuf, acc, semg0, semg1, sems0, sems1):
        c = lax.axis_index("c")
        s = lax.axis_index("s")
        base0 = _wid(c, s) * EW
        pltpu.sync_copy(den_hbm, denl)
        pltpu.sync_copy(emb_hbm, embv)
        pltpu.sync_copy(tail_hbm.at[pl.ds(base0, EW)], tailall)
        z16 = jnp.zeros((16,), jnp.float32)
        for zr in range(16):
            for zk in range(8):
                zbuf[zr, pl.ds(zk * 16, 16)] = z16

        def zinit(i, _):
            pltpu.sync_copy(zbuf, acc.at[pl.ds(s * RT + i * 16, 16)])
            return 0

        lax.fori_loop(0, RT // 16, zinit, 0)
        plsc.subcore_barrier()
        semg = (semg0, semg1)
        sems = (sems0, sems1)

        def fire_small(j, slot):
            base = base0 + j * CH
            pltpu.async_copy(
                head_hbm.at[pl.ds(base, CH)], headv.at[slot], sems[slot]
            )
            pltpu.async_copy(
                type_hbm.at[pl.ds(base, CH)], typev.at[slot, pl.ds(0, CH)],
                sems[slot],
            )
            pltpu.async_copy(
                ex_hbm.at[pl.ds(base, CH)], exv.at[slot], sems[slot]
            )

        def drain_small(slot):
            pltpu.make_async_copy(
                head_hbm.at[pl.ds(0, CH)], headv.at[slot], sems[slot]
            ).wait()
            pltpu.make_async_copy(
                type_hbm.at[pl.ds(0, CH)], typev.at[slot, pl.ds(0, CH)],
                sems[slot],
            ).wait()
            pltpu.make_async_copy(
                ex_hbm.at[pl.ds(0, CH)], exv.at[slot], sems[slot]
            ).wait()

        def fire_gather(j, slot):
            pltpu.async_copy(
                agg_hbm.at[tailall.at[pl.ds(j * CH, CH)]], abuf.at[slot],
                semg[slot],
            )

        def drain_gather(slot):
            pltpu.make_async_copy(
                agg_hbm.at[pl.ds(0, CH)], abuf.at[slot], semg[slot]
            ).wait()

        def compute(j, slot):
            ab = abuf.at[slot]
            hv = headv.at[slot]
            tv = typev.at[slot]
            ev = exv.at[slot]
            for g in range(GP):
                h = hv[pl.ds(g * 16, 16)]
                ex = ev[pl.ds(g * 16, 16)]
                dg = plsc.load_gather(denl, [h])
                attnv[pl.ds(g * 16, 16)] = ex / jnp.maximum(dg, EPS)

            def edge(e, _):
                ty = tv[pl.ds(e, 16)][0]
                at = attnv[pl.ds(e, 16)][0]
                for k in range(8):
                    ab[e, pl.ds(k * 16, 16)] = (
                        ab[e, pl.ds(k * 16, 16)]
                        * embv[ty, pl.ds(k * 16, 16)]
                        * at
                    )
                return 0

            lax.fori_loop(0, CH, edge, 0)
            pltpu.sync_copy(ab, acc.at[hv], add=True)

        fire_small(0, 0)
        fire_small(1, 1)
        fire_gather(0, 0)
        fire_gather(1, 1)

        def pair(jj, _):
            for slot in range(2):
                j = jj * 2 + slot
                drain_small(slot)
                drain_gather(slot)
                compute(j, slot)

                @pl.when(j + 2 < NCHUNK)
                def _():
                    fire_small(j + 2, slot)
                    fire_gather(j + 2, slot)
            return 0

        lax.fori_loop(0, NCHUNK // 2, pair, 0)
        drain_small(0)
        drain_gather(0)
        compute(NCHUNK - 1, 0)
        plsc.subcore_barrier()
        pltpu.sync_copy(acc.at[pl.ds(s * RT, RT)], sum_hbm.at[c, pl.ds(s * RT, RT)])

    return stage_d


def _stage_den(denpart):
    def body(d_ref, o_ref):
        o_ref[...] = d_ref[0] + d_ref[1]

    return pl.pallas_call(
        body,
        grid=(1,),
        in_specs=[pl.BlockSpec((NC, NP), lambda i: (0, 0))],
        out_specs=pl.BlockSpec((NP,), lambda i: (0,)),
        out_shape=jax.ShapeDtypeStruct((NP,), jnp.float32),
    )(denpart)


def _stage_mean_norm(sumpart):
    def body(s0_ref, s1_ref, agg_ref):
        summed = s0_ref[0] + s1_ref[0]
        nrm = jnp.sqrt(jnp.sum(summed * summed, axis=1, keepdims=True))
        agg_ref[...] = summed / jnp.maximum(nrm, EPS)

    nb = 1024
    return pl.pallas_call(
        body,
        grid=(NP // nb,),
        in_specs=[
            pl.BlockSpec((1, nb, D), lambda i: (0, i, 0)),
            pl.BlockSpec((1, nb, D), lambda i: (1, i, 0)),
        ],
        out_specs=pl.BlockSpec((nb, D), lambda i: (i, 0)),
        out_shape=jax.ShapeDtypeStruct((NP, D), jnp.float32),
    )(sumpart, sumpart)


def _stage_final(agg1, agg2, ent):
    def body(a1, a2, e, o):
        o[...] = a1[...] + a2[...] + 2.0 * e[...]

    nb = 256
    return pl.pallas_call(
        body,
        grid=(NP // nb,),
        in_specs=[pl.BlockSpec((nb, D), lambda i: (i, 0))] * 3,
        out_specs=pl.BlockSpec((nb, D), lambda i: (i, 0)),
        out_shape=jax.ShapeDtypeStruct((NP, D), jnp.float32),
    )(agg1, agg2, ent)


@functools.cache
def _build():
    return _make_stage_b(), _make_stage_c(), _make_stage_d()


def kernel(entity_emb, edge_index, edge_type, edge_emb, q_w, k_w, mess_dropout):
    stage_b, stage_c, stage_d = _build()
    head = edge_index[0]
    tail = edge_index[1]
    etype = edge_type.astype(jnp.int32)
    ent_p = jnp.pad(entity_emb, ((0, NP - N), (0, 0)))
    agg = ent_p
    aggs = []
    for _hop in range(2):
        p_arr, t2f = _stage_a(agg, edge_emb, q_w, k_w)
        att, maxpart = stage_b(p_arr, t2f, head, tail, etype)
        ex, denpart = stage_c(att, head, maxpart)
        den = _stage_den(denpart)
        sumpart = stage_d(ex, head, tail, etype, den, edge_emb, agg)
        aggn = _stage_mean_norm(sumpart)
        aggs.append(aggn)
        agg = aggn
    kg = _stage_final(aggs[0], aggs[1], ent_p)
    return kg[:N]
